# baseline (speedup 1.0000x reference)
import dataclasses

import jax
import jax.numpy as jnp
from jax import lax
from jax.experimental import pallas as pl
from jax.experimental.pallas import tpu as pltpu
from jax.experimental.pallas import tpu_sc as plsc

N = 100000
D = 128
B_SEG = 512
L = 16
NCORES = 2
NSUB = 16
NW = NCORES * NSUB
RPW = N // NW
BLK = 96
NBUF = 8
NSLOT = 8

ID_CHUNK = 6256
ID_MAIN = 6160
ID_TAIL = ID_CHUNK - ID_MAIN
NVEC_MAIN = ID_MAIN // L
NVEC_FULL = ID_CHUNK // L


def _sc_body(x_hbm, ids_hbm, out_hbm,
             ids_buf, cnt_ref, merged_ref, ends_ref,
             buf_0, buf_1, buf_2, buf_3, buf_4, buf_5, buf_6, buf_7,
             row_stage, zero_row,
             sem_0, sem_1, sem_2, sem_3, sem_4, sem_5, sem_6, sem_7,
             out_sem, shared_cnt):
    bufs = (buf_0, buf_1, buf_2, buf_3, buf_4, buf_5, buf_6, buf_7)
    sems = (sem_0, sem_1, sem_2, sem_3, sem_4, sem_5, sem_6, sem_7)
    c = lax.axis_index("c")
    s_sub = lax.axis_index("s")
    w = c * NSUB + s_sub
    base = w * RPW
    base8 = (base // 8) * 8

    def dma_start(b, buf, sem):
        p = base8 + b * BLK
        ld = pl.multiple_of(jnp.minimum(p, N - BLK), 8)
        pltpu.async_copy(x_hbm.at[pl.ds(ld, BLK)], buf, sem)

    def dma_wait(buf, sem):
        pltpu.make_async_copy(x_hbm.at[pl.ds(0, BLK)], buf, sem).wait()

    ib = pl.multiple_of(s_sub * ID_CHUNK, 8)
    ids_cp = pltpu.async_copy(ids_hbm.at[pl.ds(ib, ID_MAIN)],
                              ids_buf.at[pl.ds(0, ID_MAIN)], out_sem)

    @pl.when(s_sub < NSUB - 1)
    def _():
        ib2 = pl.multiple_of(s_sub * ID_CHUNK + ID_MAIN, 8)
        pltpu.async_copy(ids_hbm.at[pl.ds(ib2, ID_TAIL)],
                         ids_buf.at[pl.ds(ID_MAIN, ID_TAIL)], out_sem)

    for r in range(NBUF - 1):
        dma_start(r, bufs[r], sems[r])

    @pl.loop(0, B_SEG // L)
    def _(j):
        off = pl.multiple_of(j * L, L)
        cnt_ref[pl.ds(off, L)] = jnp.zeros((L,), jnp.int32)

    ids_cp.wait()

    @pl.when(s_sub < NSUB - 1)
    def _():
        pltpu.make_async_copy(ids_hbm.at[pl.ds(0, ID_TAIL)],
                              ids_buf.at[pl.ds(ID_MAIN, ID_TAIL)],
                              out_sem).wait()

    ones = jnp.ones((L,), jnp.int32)
    nvec = jnp.where(s_sub < NSUB - 1, NVEC_FULL, NVEC_MAIN)

    def _hist(i, carry):
        off = pl.multiple_of(i * L, L)
        v = ids_buf[pl.ds(off, L)]
        plsc.addupdate_scatter(cnt_ref, [v], ones)
        return carry

    lax.fori_loop(0, nvec, _hist, 0)

    pltpu.sync_copy(cnt_ref, shared_cnt.at[s_sub])
    plsc.subcore_barrier()
    pltpu.sync_copy(shared_cnt, merged_ref)

    def _ends(j, carry):
        off = pl.multiple_of(j * L, L)
        acc = jnp.zeros((L,), jnp.int32)
        for r in range(NSUB):
            acc = acc + merged_ref[r, pl.ds(off, L)]
        e = plsc.cumsum(acc) + carry
        ends_ref[pl.ds(off, L)] = e
        return carry + jnp.sum(acc)

    lax.fori_loop(0, B_SEG // L, _ends, jnp.int32(0))

    def count_lt(val):
        def _cl(j, cnt):
            off = pl.multiple_of(j * L, L)
            m = ends_ref[pl.ds(off, L)] < val
            return cnt + plsc.all_reduce_population_count(m)[0]

        return lax.fori_loop(0, B_SEG // L, _cl, jnp.int32(0))

    s_first = jnp.where(w > 0, 1 + count_lt(base), 0)
    s_end = jnp.where(w < NW - 1, 1 + count_lt(base + RPW), B_SEG)
    s_end = jnp.maximum(s_end, s_first)
    nseg = s_end - s_first

    iota = lax.iota(jnp.int32, L)

    def seg_bounds(ck):
        idx = jnp.clip(ck - 1 + iota, 0, B_SEG - 1)
        g = plsc.load_gather(ends_ref, [idx])
        st_k = jnp.where(ck > 0, g[0], 0)
        return st_k, g[1]

    row_lo0, _ = seg_bounds(s_first)
    row_lo = jnp.where(s_first > 0, row_lo0, 0)
    lastst, row_hi0 = seg_bounds(jnp.maximum(s_end - 1, 0))
    row_hi = jnp.where(nseg > 0, row_hi0, row_lo)
    nb = jnp.where(nseg > 0, (row_hi - base8 + BLK - 1) // BLK, 0)
    nbp = jnp.maximum(nb, NBUF - 1)

    for j in range(3 * D // L):
        zero_row[pl.ds(pl.multiple_of(j * L, L), L)] = jnp.zeros((L,), jnp.float32)

    def out_drain_one():
        pltpu.make_async_copy(out_hbm.at[pl.ds(0, 3 * D)], zero_row,
                              out_sem).wait()

    def emit_row(ck, vecs):
        i = ck - s_first

        @pl.when(jnp.logical_and(i >= NSLOT,
                                 jnp.bitwise_and(i, NSLOT - 1) == 0))
        def _():
            @pl.loop(0, NSLOT)
            def _(_):
                out_drain_one()

        off = pl.multiple_of(ck * (3 * D), 8)
        if vecs is None:
            pltpu.async_copy(zero_row, out_hbm.at[pl.ds(off, 3 * D)], out_sem)
        else:
            slot = jnp.bitwise_and(i, NSLOT - 1)
            for j, v in enumerate(vecs):
                row_stage[slot, pl.ds(j * L, L)] = v
            pltpu.async_copy(row_stage.at[slot],
                             out_hbm.at[pl.ds(off, 3 * D)], out_sem)

    zeros = jnp.zeros((L,), jnp.float32)
    ninf = jnp.full((L,), -jnp.inf, jnp.float32)
    sums0 = (zeros,) * (D // L)
    maxs0 = (ninf,) * (D // L)

    def process_block(b, buf, carry):
        cur_k, sums, maxs = carry
        p = base8 + b * BLK
        ld = jnp.minimum(p, N - BLK)
        blo = jnp.maximum(p, row_lo)
        bhi = jnp.minimum(p + BLK, row_hi)

        def cond(cst):
            return cst[0] < bhi

        def body(cst):
            pos, ck, csums, cmaxs = cst
            st_k, en_k = seg_bounds(ck)
            run_hi = jnp.minimum(bhi, en_k)

            def _row(r, rc):
                rsums, rmaxs = rc
                nsums, nmaxs = [], []
                for j in range(D // L):
                    xv = buf[r, pl.ds(j * L, L)]
                    nsums.append(rsums[j] + xv)
                    nmaxs.append(jnp.maximum(rmaxs[j], xv))
                return (tuple(nsums), tuple(nmaxs))

            csums, cmaxs = lax.fori_loop(pos - ld, run_hi - ld, _row,
                                         (csums, cmaxs))
            finished = en_k <= bhi

            @pl.when(finished)
            def _():
                cnt = en_k - st_k
                cntv = jnp.full((L,), cnt.astype(jnp.float32))
                inv = jnp.ones((L,), jnp.float32) / jnp.maximum(cntv, 1.0)
                nonempty = cnt > 0
                vecs = ([jnp.where(nonempty, m, 0.0) for m in cmaxs]
                        + list(csums)
                        + [s * inv for s in csums])
                emit_row(ck, vecs)

            ck2 = jnp.where(finished, ck + 1, ck)
            csums = tuple(jnp.where(finished, 0.0, v) for v in csums)
            cmaxs = tuple(jnp.where(finished, -jnp.inf, v) for v in cmaxs)
            return (run_hi, ck2, csums, cmaxs)

        out = lax.while_loop(cond, body, (blo, cur_k, sums, maxs))
        return (out[1], out[2], out[3])

    def outer(i, carry):
        for r in range(NBUF):
            b = NBUF * i + r

            @pl.when(b < nbp)
            def _(r=r):
                dma_wait(bufs[r], sems[r])

            @pl.when(b + NBUF - 1 < nb)
            def _(b=b, r=r):
                dma_start(b + NBUF - 1, bufs[(r + NBUF - 1) % NBUF],
                          sems[(r + NBUF - 1) % NBUF])

            carry = process_block(b, bufs[r], carry)
        return carry

    carry = lax.fori_loop(0, (nbp + NBUF - 1) // NBUF, outer,
                          (s_first, sums0, maxs0))
    ck_final = carry[0]

    def _empties(ck, carry2):
        emit_row(ck, None)
        return carry2

    lax.fori_loop(ck_final, s_end, _empties, 0)

    remaining = jnp.where(nseg > 0,
                          jnp.bitwise_and(nseg - 1, NSLOT - 1) + 1, 0)

    def _drain(i, carry2):
        out_drain_one()
        return carry2

    lax.fori_loop(0, remaining, _drain, 0)


@jax.jit
def _diffpool_readout(x, ids):
    mesh = plsc.VectorSubcoreMesh(core_axis_name="c", subcore_axis_name="s")
    cp = pltpu.CompilerParams()
    if "needs_layout_passes" in pltpu.CompilerParams.__dataclass_fields__:
        cp = dataclasses.replace(cp, needs_layout_passes=False)
    f = pl.kernel(
        _sc_body,
        out_type=jax.ShapeDtypeStruct((B_SEG * 3 * D,), jnp.float32),
        mesh=mesh,
        scratch_types=[
            pltpu.VMEM((ID_CHUNK,), jnp.int32),
            pltpu.VMEM((B_SEG,), jnp.int32),
            pltpu.VMEM((NSUB, B_SEG), jnp.int32),
            pltpu.VMEM((B_SEG,), jnp.int32),
            pltpu.VMEM((BLK, D), jnp.float32),
            pltpu.VMEM((BLK, D), jnp.float32),
            pltpu.VMEM((BLK, D), jnp.float32),
            pltpu.VMEM((BLK, D), jnp.float32),
            pltpu.VMEM((BLK, D), jnp.float32),
            pltpu.VMEM((BLK, D), jnp.float32),
            pltpu.VMEM((BLK, D), jnp.float32),
            pltpu.VMEM((BLK, D), jnp.float32),
            pltpu.VMEM((NSLOT, 3 * D), jnp.float32),
            pltpu.VMEM((3 * D,), jnp.float32),
            pltpu.SemaphoreType.DMA,
            pltpu.SemaphoreType.DMA,
            pltpu.SemaphoreType.DMA,
            pltpu.SemaphoreType.DMA,
            pltpu.SemaphoreType.DMA,
            pltpu.SemaphoreType.DMA,
            pltpu.SemaphoreType.DMA,
            pltpu.SemaphoreType.DMA,
            pltpu.SemaphoreType.DMA,
            pltpu.VMEM_SHARED((NSUB, B_SEG), jnp.int32),
        ],
        compiler_params=cp,
    )
    return f(x, ids).reshape(B_SEG, 3 * D)


def kernel(x, segment_ids):
    return _diffpool_readout(x, segment_ids.astype(jnp.int32))


# speedup vs baseline: 1.0403x; 1.0403x over previous
import dataclasses

import jax
import jax.numpy as jnp
from jax import lax
from jax.experimental import pallas as pl
from jax.experimental.pallas import tpu as pltpu
from jax.experimental.pallas import tpu_sc as plsc

N = 100000
D = 128
B_SEG = 512
L = 16
NCORES = 2
NSUB = 16
NW = NCORES * NSUB
SEGS_PER_W = B_SEG // NW
BLK = 160
NBUF = 4

ID_CHUNK = 6256
ID_MAIN = 6160
ID_TAIL = ID_CHUNK - ID_MAIN
NVEC_MAIN = ID_MAIN // L
NVEC_FULL = ID_CHUNK // L


def _sc_body(x_hbm, ids_hbm, out_hbm,
             ids_buf, cnt_ref, merged_ref, ends_ref,
             buf_0, buf_1, buf_2, buf_3, row_ref,
             bnd_smem, sem_0, sem_1, sem_2, sem_3, shared_cnt):
    bufs = (buf_0, buf_1, buf_2, buf_3)
    sems = (sem_0, sem_1, sem_2, sem_3)
    c = lax.axis_index("c")
    s_sub = lax.axis_index("s")
    w = c * NSUB + s_sub

    @pl.loop(0, B_SEG // L)
    def _(j):
        off = pl.multiple_of(j * L, L)
        cnt_ref[pl.ds(off, L)] = jnp.zeros((L,), jnp.int32)

    base = pl.multiple_of(s_sub * ID_CHUNK, 8)
    pltpu.sync_copy(ids_hbm.at[pl.ds(base, ID_MAIN)], ids_buf.at[pl.ds(0, ID_MAIN)])

    @pl.when(s_sub < NSUB - 1)
    def _():
        base2 = pl.multiple_of(s_sub * ID_CHUNK + ID_MAIN, 8)
        pltpu.sync_copy(ids_hbm.at[pl.ds(base2, ID_TAIL)],
                        ids_buf.at[pl.ds(ID_MAIN, ID_TAIL)])

    ones = jnp.ones((L,), jnp.int32)
    nvec = jnp.where(s_sub < NSUB - 1, NVEC_FULL, NVEC_MAIN)

    def _hist(i, carry):
        off = pl.multiple_of(i * L, L)
        v = ids_buf[pl.ds(off, L)]
        plsc.addupdate_scatter(cnt_ref, [v], ones)
        return carry

    lax.fori_loop(0, nvec, _hist, 0)

    pltpu.sync_copy(cnt_ref, shared_cnt.at[s_sub])
    plsc.subcore_barrier()
    pltpu.sync_copy(shared_cnt, merged_ref)

    def _ends(j, carry):
        off = pl.multiple_of(j * L, L)
        acc = jnp.zeros((L,), jnp.int32)
        for r in range(NSUB):
            acc = acc + merged_ref[r, pl.ds(off, L)]
        e = plsc.cumsum(acc) + carry
        ends_ref[pl.ds(off, L)] = e
        return carry + jnp.sum(acc)

    lax.fori_loop(0, B_SEG // L, _ends, jnp.int32(0))

    ev = ends_ref[pl.ds(pl.multiple_of(w * SEGS_PER_W, L), L)]
    pv = ends_ref[pl.ds(pl.multiple_of(jnp.maximum(w - 1, 0) * SEGS_PER_W, L), L)]
    st_first = jnp.where(w > 0, pv[L - 1], 0)
    bnd_smem[0] = st_first
    for k in range(SEGS_PER_W):
        bnd_smem[k + 1] = ev[k]
    stream_hi = ev[SEGS_PER_W - 1]
    st8 = (st_first // 8) * 8
    nb = (stream_hi - st8 + BLK - 1) // BLK

    def dma_start(b, buf, sem):
        p = st8 + b * BLK
        ld = pl.multiple_of(jnp.minimum(p, N - BLK), 8)
        pltpu.async_copy(x_hbm.at[pl.ds(ld, BLK)], buf, sem)

    def dma_wait(buf, sem):
        pltpu.make_async_copy(x_hbm.at[pl.ds(0, BLK)], buf, sem).wait()

    for r in range(NBUF - 1):
        @pl.when(r < nb)
        def _(r=r):
            dma_start(r, bufs[r], sems[r])

    zeros = jnp.zeros((L,), jnp.float32)
    ninf = jnp.full((L,), -jnp.inf, jnp.float32)
    sums0 = (zeros,) * (D // L)
    maxs0 = (ninf,) * (D // L)

    def process_block(b, buf, carry):
        cur_k, sums, maxs = carry
        p = st8 + b * BLK
        ld = jnp.minimum(p, N - BLK)
        blo = jnp.maximum(p, st_first)
        bhi = jnp.minimum(p + BLK, stream_hi)

        def cond(c):
            return c[0] < bhi

        def body(c):
            pos, ck, csums, cmaxs = c
            st_k = bnd_smem[ck]
            en_k = bnd_smem[ck + 1]
            run_hi = jnp.minimum(bhi, en_k)

            def _row(r, rc):
                rsums, rmaxs = rc
                nsums, nmaxs = [], []
                for j in range(D // L):
                    xv = buf[r, pl.ds(j * L, L)]
                    nsums.append(rsums[j] + xv)
                    nmaxs.append(jnp.maximum(rmaxs[j], xv))
                return (tuple(nsums), tuple(nmaxs))

            csums, cmaxs = lax.fori_loop(pos - ld, run_hi - ld, _row,
                                         (csums, cmaxs))
            finished = en_k <= bhi

            @pl.when(finished)
            def _():
                cnt = en_k - st_k
                cntv = jnp.full((L,), cnt.astype(jnp.float32))
                inv = jnp.ones((L,), jnp.float32) / jnp.maximum(cntv, 1.0)
                nonempty = cnt > 0
                for j in range(D // L):
                    mx = jnp.where(nonempty, cmaxs[j], 0.0)
                    row_ref[ck, pl.ds(j * L, L)] = mx
                    row_ref[ck, pl.ds(D + j * L, L)] = csums[j]
                    row_ref[ck, pl.ds(2 * D + j * L, L)] = csums[j] * inv

            ck2 = jnp.where(finished, ck + 1, ck)
            csums = tuple(jnp.where(finished, 0.0, v) for v in csums)
            cmaxs = tuple(jnp.where(finished, -jnp.inf, v) for v in cmaxs)
            return (run_hi, ck2, csums, cmaxs)

        out = lax.while_loop(cond, body, (blo, cur_k, sums, maxs))
        return (out[1], out[2], out[3])

    def outer(i, carry):
        for r in range(NBUF):
            b = NBUF * i + r

            @pl.when(b < nb)
            def _(r=r):
                dma_wait(bufs[r], sems[r])

            @pl.when(b + NBUF - 1 < nb)
            def _(b=b, r=r):
                dma_start(b + NBUF - 1, bufs[(r + NBUF - 1) % NBUF],
                          sems[(r + NBUF - 1) % NBUF])

            carry = process_block(b, bufs[r], carry)
        return carry

    carry = lax.fori_loop(0, (nb + NBUF - 1) // NBUF, outer,
                          (jnp.int32(0), sums0, maxs0))
    cur_k_final = carry[0]

    for k in range(SEGS_PER_W):
        @pl.when(k >= cur_k_final)
        def _():
            for j in range(D // L):
                row_ref[k, pl.ds(j * L, L)] = zeros
                row_ref[k, pl.ds(D + j * L, L)] = zeros
                row_ref[k, pl.ds(2 * D + j * L, L)] = zeros

    out_base = pl.multiple_of(w * SEGS_PER_W, 8)
    pltpu.sync_copy(row_ref, out_hbm.at[pl.ds(out_base, SEGS_PER_W)])


@jax.jit
def _diffpool_readout(x, ids):
    mesh = plsc.VectorSubcoreMesh(core_axis_name="c", subcore_axis_name="s")
    cp = pltpu.CompilerParams()
    if "needs_layout_passes" in pltpu.CompilerParams.__dataclass_fields__:
        cp = dataclasses.replace(cp, needs_layout_passes=False)
    f = pl.kernel(
        _sc_body,
        out_type=jax.ShapeDtypeStruct((B_SEG, 3 * D), jnp.float32),
        mesh=mesh,
        scratch_types=[
            pltpu.VMEM((ID_CHUNK,), jnp.int32),
            pltpu.VMEM((B_SEG,), jnp.int32),
            pltpu.VMEM((NSUB, B_SEG), jnp.int32),
            pltpu.VMEM((B_SEG,), jnp.int32),
            pltpu.VMEM((BLK, D), jnp.float32),
            pltpu.VMEM((BLK, D), jnp.float32),
            pltpu.VMEM((BLK, D), jnp.float32),
            pltpu.VMEM((BLK, D), jnp.float32),
            pltpu.VMEM((SEGS_PER_W, 3 * D), jnp.float32),
            pltpu.SMEM((SEGS_PER_W + 1,), jnp.int32),
            pltpu.SemaphoreType.DMA,
            pltpu.SemaphoreType.DMA,
            pltpu.SemaphoreType.DMA,
            pltpu.SemaphoreType.DMA,
            pltpu.VMEM_SHARED((NSUB, B_SEG), jnp.int32),
        ],
        compiler_params=cp,
    )
    return f(x, ids)


def kernel(x, segment_ids):
    return _diffpool_readout(x, segment_ids.astype(jnp.int32))
